# baseline (speedup 1.0000x reference)
import functools

import jax
import jax.numpy as jnp
from jax import lax
from jax.experimental import pallas as pl
from jax.experimental.pallas import tpu as pltpu
from jax.experimental.pallas import tpu_sc as plsc

B = 4096
F = 26
V = 1000
D = 128
L = F * D

_NC = 2
_NS = 16
_NW = _NC * _NS
_ROWS = B * F
_RPW = _ROWS // _NW
_CH = 128
_NCHUNK = _RPW // _CH


def _gather_body(idx_hbm, table_hbm, out_hbm, idx_v, buf, sem):
    c = lax.axis_index("c")
    s = lax.axis_index("s")
    wid = s * _NC + c
    base = wid * _RPW
    pltpu.sync_copy(idx_hbm.at[wid], idx_v)

    def chunk(i, carry):
        pltpu.async_copy(table_hbm.at[idx_v.at[i]], buf, sem).wait()
        pltpu.sync_copy(buf, out_hbm.at[pl.ds(base + i * _CH, _CH)])
        return carry

    lax.fori_loop(0, _NCHUNK, chunk, 0)


@functools.cache
def _gather():
    return functools.partial(
        pl.kernel,
        out_type=jax.ShapeDtypeStruct((_ROWS, D), jnp.float32),
        mesh=plsc.VectorSubcoreMesh(core_axis_name="c", subcore_axis_name="s"),
        scratch_types=[
            pltpu.VMEM((_NCHUNK, _CH), jnp.int32),
            pltpu.VMEM((_CH, D), jnp.float32),
            pltpu.SemaphoreType.DMA,
        ],
    )(_gather_body)


_BB = 256
_GRID = B // _BB


def _mlp_body(r_ref, w10, b10, w20, b20, w11, b11, w21, b21, w12, b12,
              w22, b22, wd, bd, out_ref):
    bf = jnp.bfloat16
    r = r_ref[...]
    for w1, b1, w2, b2 in ((w10, b10, w20, b20),
                           (w11, b11, w21, b21),
                           (w12, b12, w22, b22)):
        h = jnp.dot(r.astype(bf), w1[...],
                    preferred_element_type=jnp.float32) + b1[...]
        h = jnp.maximum(h, 0.0)
        h = jnp.dot(h.astype(bf), w2[...],
                    preferred_element_type=jnp.float32) + b2[...]
        r = jnp.maximum(r + h, 0.0)
    logit = jnp.dot(r.astype(bf), wd[...],
                    preferred_element_type=jnp.float32) + bd[...]
    out_ref[...] = jax.nn.sigmoid(logit)


def _resident(shape):
    return pl.BlockSpec(shape, lambda i: (0,) * len(shape))


def _mlp(r, w10, b10, w20, b20, w11, b11, w21, b21, w12, b12, w22, b22,
         wd, bd):
    u = w10.shape[1]
    in_specs = [pl.BlockSpec((_BB, L), lambda i: (i, 0))]
    for _ in range(3):
        in_specs += [_resident((L, u)), _resident((1, u)),
                     _resident((u, L)), _resident((1, L))]
    in_specs += [_resident((L, 1)), _resident((1, 1))]
    return pl.pallas_call(
        _mlp_body,
        grid=(_GRID,),
        in_specs=in_specs,
        out_specs=pl.BlockSpec((_BB, 1), lambda i: (i, 0)),
        out_shape=jax.ShapeDtypeStruct((B, 1), jnp.float32),
        compiler_params=pltpu.CompilerParams(
            dimension_semantics=("arbitrary",),
        ),
    )(r, w10, b10, w20, b20, w11, b11, w21, b21, w12, b12, w22, b22, wd, bd)


def kernel(inputs, tables, W1_0, b1_0, W2_0, b2_0, W1_1, b1_1, W2_1, b2_1,
           W1_2, b1_2, W2_2, b2_2, Wd, bd):
    table_flat = tables.reshape(F * V, D)
    offs = (jnp.arange(F, dtype=jnp.int32) * V)[None, :]
    flat_idx = (inputs.astype(jnp.int32) + offs).reshape(_NW, _NCHUNK, _CH)
    r = _gather()(flat_idx, table_flat).reshape(B, L)
    bf = jnp.bfloat16
    return _mlp(
        r,
        W1_0.astype(bf), b1_0.reshape(1, -1), W2_0.astype(bf), b2_0.reshape(1, -1),
        W1_1.astype(bf), b1_1.reshape(1, -1), W2_1.astype(bf), b2_1.reshape(1, -1),
        W1_2.astype(bf), b1_2.reshape(1, -1), W2_2.astype(bf), b2_2.reshape(1, -1),
        Wd.astype(bf), bd.reshape(1, 1),
    )


# speedup vs baseline: 1.1690x; 1.1690x over previous
import functools

import jax
import jax.numpy as jnp
from jax import lax
from jax.experimental import pallas as pl
from jax.experimental.pallas import tpu as pltpu
from jax.experimental.pallas import tpu_sc as plsc

B = 4096
F = 26
V = 1000
D = 128
L = F * D

_NC = 2
_NS = 16
_NW = _NC * _NS
_ROWS = B * F
_RPW = _ROWS // _NW
_CH = 128
_NCHUNK = _RPW // _CH


def _gather_body(idx_hbm, table_hbm, out_hbm, idx_v, buf0, buf1, sem0, sem1):
    c = lax.axis_index("c")
    s = lax.axis_index("s")
    wid = s * _NC + c
    base = wid * _RPW
    pltpu.sync_copy(idx_hbm.at[wid], idx_v)
    pltpu.async_copy(table_hbm.at[idx_v.at[0]], buf0, sem0)

    def pair(g, carry):
        i0 = 2 * g
        i1 = i0 + 1
        pltpu.async_copy(table_hbm.at[idx_v.at[i1]], buf1, sem1)
        pltpu.make_async_copy(table_hbm.at[idx_v.at[i0]], buf0, sem0).wait()
        pltpu.sync_copy(buf0, out_hbm.at[pl.ds(base + i0 * _CH, _CH)])

        @pl.when(i1 + 1 < _NCHUNK)
        def _():
            pltpu.async_copy(table_hbm.at[idx_v.at[i1 + 1]], buf0, sem0)

        pltpu.make_async_copy(table_hbm.at[idx_v.at[i1]], buf1, sem1).wait()
        pltpu.sync_copy(buf1, out_hbm.at[pl.ds(base + i1 * _CH, _CH)])
        return carry

    lax.fori_loop(0, _NCHUNK // 2, pair, 0)


@functools.cache
def _gather():
    return functools.partial(
        pl.kernel,
        out_type=jax.ShapeDtypeStruct((_ROWS, D), jnp.float32),
        mesh=plsc.VectorSubcoreMesh(core_axis_name="c", subcore_axis_name="s"),
        scratch_types=[
            pltpu.VMEM((_NCHUNK, _CH), jnp.int32),
            pltpu.VMEM((_CH, D), jnp.float32),
            pltpu.VMEM((_CH, D), jnp.float32),
            pltpu.SemaphoreType.DMA,
            pltpu.SemaphoreType.DMA,
        ],
    )(_gather_body)


_BB = 256
_GRID = B // _BB


def _mlp_body(r_ref, w10, b10, w20, b20, w11, b11, w21, b21, w12, b12,
              w22, b22, wd, bd, out_ref):
    r = r_ref[...]
    for w1, b1, w2, b2 in ((w10, b10, w20, b20),
                           (w11, b11, w21, b21),
                           (w12, b12, w22, b22)):
        h = jnp.dot(r, w1[...], preferred_element_type=jnp.float32) + b1[...]
        h = jnp.maximum(h, 0.0)
        h = jnp.dot(h, w2[...], preferred_element_type=jnp.float32) + b2[...]
        r = jnp.maximum(r + h, 0.0)
    logit = jnp.dot(r, wd[...], preferred_element_type=jnp.float32) + bd[...]
    out_ref[...] = jax.nn.sigmoid(logit)


def _resident(shape):
    return pl.BlockSpec(shape, lambda i: (0,) * len(shape))


def _mlp(r, w10, b10, w20, b20, w11, b11, w21, b21, w12, b12, w22, b22,
         wd, bd):
    u = w10.shape[1]
    in_specs = [pl.BlockSpec((_BB, L), lambda i: (i, 0))]
    for _ in range(3):
        in_specs += [_resident((L, u)), _resident((1, u)),
                     _resident((u, L)), _resident((1, L))]
    in_specs += [_resident((L, 1)), _resident((1, 1))]
    return pl.pallas_call(
        _mlp_body,
        grid=(_GRID,),
        in_specs=in_specs,
        out_specs=pl.BlockSpec((_BB, 1), lambda i: (i, 0)),
        out_shape=jax.ShapeDtypeStruct((B, 1), jnp.float32),
        compiler_params=pltpu.CompilerParams(
            dimension_semantics=("arbitrary",),
        ),
    )(r, w10, b10, w20, b20, w11, b11, w21, b21, w12, b12, w22, b22, wd, bd)


def kernel(inputs, tables, W1_0, b1_0, W2_0, b2_0, W1_1, b1_1, W2_1, b2_1,
           W1_2, b1_2, W2_2, b2_2, Wd, bd):
    table_flat = tables.reshape(F * V, D)
    offs = (jnp.arange(F, dtype=jnp.int32) * V)[None, :]
    flat_idx = (inputs.astype(jnp.int32) + offs).reshape(_NW, _NCHUNK, _CH)
    r = _gather()(flat_idx, table_flat).reshape(B, L)
    return _mlp(
        r,
        W1_0, b1_0.reshape(1, -1), W2_0, b2_0.reshape(1, -1),
        W1_1, b1_1.reshape(1, -1), W2_1, b2_1.reshape(1, -1),
        W1_2, b1_2.reshape(1, -1), W2_2, b2_2.reshape(1, -1),
        Wd, bd.reshape(1, 1),
    )
